# baseline (speedup 1.0000x reference)
import functools

import numpy as np
import jax
import jax.numpy as jnp
from jax.experimental import pallas as pl
from jax.experimental.pallas import tpu as pltpu

_CAPACITY_FACTOR = 1.2
_ROW_BLK = 256


def _bits_masked(k0, k1, n, n_max):
    ji = jnp.arange(n_max, dtype=jnp.int32)
    half = (n + 1) // 2
    x0 = ji.astype(jnp.uint32)
    x1 = jnp.where(ji < (n // 2), ji + half, 0).astype(jnp.uint32)
    ks2 = k0 ^ k1 ^ jnp.uint32(0x1BD11BDA)
    ks = (k0, k1, ks2)
    v0 = x0 + ks[0]
    v1 = x1 + ks[1]
    rotations = ((13, 15, 26, 6), (17, 29, 16, 24))
    for i in range(5):
        for r in rotations[i % 2]:
            v0 = v0 + v1
            v1 = (v1 << jnp.uint32(r)) | (v1 >> jnp.uint32(32 - r))
            v1 = v0 ^ v1
        v0 = v0 + ks[(i + 1) % 3]
        v1 = v1 + ks[(i + 2) % 3] + jnp.uint32(i + 1)
    lo = v1[jnp.clip(ji - half, 0, n_max - 1)]
    return jnp.where(ji < half, v0, lo)


def _round_key_data(E, max_rounds):
    keys = [jax.random.fold_in(jax.random.key(1), i) for i in range(E)]
    rows = []
    for _ in range(max_rounds):
        subs = []
        for i in range(E):
            keys[i], sub = jax.random.split(keys[i])
            subs.append(jax.random.key_data(sub))
        rows.append(jnp.stack(subs))
    return jnp.stack(rows)


def _expert_matmul_kernel(x_ref, w_ref, b_ref, o_ref):
    acc = jax.lax.dot_general(
        x_ref[0], w_ref[0],
        dimension_numbers=(((1,), (1,)), ((), ())),
        preferred_element_type=jnp.float32,
    )
    o_ref[0] = acc + b_ref[0]


def _expert_matmul(xg, W_experts, b_experts, cap_pad):
    E, D = W_experts.shape[0], W_experts.shape[1]
    grid = (E, cap_pad // _ROW_BLK)
    return pl.pallas_call(
        _expert_matmul_kernel,
        grid=grid,
        in_specs=[
            pl.BlockSpec((1, _ROW_BLK, D), lambda i, c: (i, c, 0)),
            pl.BlockSpec((1, D, D), lambda i, c: (i, 0, 0)),
            pl.BlockSpec((1, 1, D), lambda i, c: (i, 0, 0)),
        ],
        out_specs=pl.BlockSpec((1, _ROW_BLK, D), lambda i, c: (i, c, 0)),
        out_shape=jax.ShapeDtypeStruct((E, cap_pad, D), jnp.float32),
    )(xg.reshape(E, cap_pad, D), W_experts, b_experts.reshape(E, 1, D))


def kernel(x, W_switch, b_switch, W_experts, b_experts):
    b, s, d = x.shape
    E = W_switch.shape[0]
    n_tok = b * s
    xf = x.reshape(-1, d)

    logits = xf @ W_switch.T + b_switch
    probs = jax.nn.softmax(logits, axis=-1)
    route_probs = jnp.max(probs, axis=-1)
    routes = jnp.argmax(probs, axis=-1).astype(jnp.int32)

    capacity = int(_CAPACITY_FACTOR * n_tok / E)
    cap_pad = ((capacity + _ROW_BLK - 1) // _ROW_BLK) * _ROW_BLK
    rounds_np = np.array([int(np.ceil(3 * np.log(max(1, t)) / np.log(2)))
                          for t in range(n_tok + 1)], dtype=np.int32)
    rounds_table = jnp.asarray(rounds_np)
    max_rounds = int(rounds_np.max())

    eids = jnp.arange(E, dtype=jnp.int32)
    counts = jnp.sum(routes[None, :] == eids[:, None], axis=1).astype(jnp.int32)
    num_rounds = rounds_table[counts]
    r_needed = jnp.max(num_rounds)

    rk = _round_key_data(E, max_rounds)
    ji = jnp.arange(n_tok, dtype=jnp.int32)

    kflat = rk.reshape(max_rounds * E, 2)
    nflat = jnp.tile(counts, max_rounds)
    bits = jax.vmap(lambda a, c, n: _bits_masked(a, c, n, n_tok))(
        kflat[:, 0], kflat[:, 1], nflat)
    active = ((ji[None, :] < nflat[:, None])
              & (jnp.repeat(jnp.arange(max_rounds, dtype=jnp.int32), E)[:, None]
                 < jnp.tile(num_rounds, max_rounds)[:, None]))
    keys = jnp.where(active, bits, jnp.uint32(0xFFFFFFFF))
    payload = jnp.broadcast_to(ji[None, :], keys.shape).astype(jnp.int32)
    _, sigma = jax.lax.sort((keys, payload), dimension=1, num_keys=1,
                            is_stable=True)

    sig = sigma.reshape(max_rounds, E, n_tok)
    r_cur = max_rounds
    while r_cur > 1:
        half = r_cur // 2
        a = sig[0:2 * half:2]
        bb = sig[1:2 * half:2]
        comp = jnp.take_along_axis(a, bb, axis=2)
        if r_cur % 2:
            comp = jnp.concatenate([comp, sig[2 * half:]], axis=0)
        sig = comp
        r_cur = half + (r_cur % 2)
    perm = sig[0]

    rowi = jnp.broadcast_to(eids[:, None], (E, n_tok))
    colj = jnp.broadcast_to(ji[None, :], (E, n_tok))
    inv = jnp.zeros((E, n_tok), jnp.int32).at[rowi, perm].set(colj)
    keep_rank = (counts[:, None] <= capacity) | (inv < capacity)

    sorted_tok = jnp.argsort(routes, stable=True).astype(jnp.int32)
    e_sorted = routes[sorted_tok]
    starts = jnp.concatenate([jnp.zeros((1,), jnp.int32),
                              jnp.cumsum(counts)[:-1].astype(jnp.int32)])
    pos = ji - starts[e_sorted]
    kept_sorted = keep_rank[e_sorted, pos]

    kept_count = jnp.minimum(counts, capacity)
    kept_before = jnp.concatenate([jnp.zeros((1,), jnp.int32),
                                   jnp.cumsum(kept_count)[:-1].astype(jnp.int32)])
    kc = jnp.cumsum(kept_sorted.astype(jnp.int32))
    slot = e_sorted * cap_pad + (kc - 1 - kept_before[e_sorted])

    d_flat = jnp.full((E * cap_pad,), n_tok, jnp.int32)
    d_flat = d_flat.at[jnp.where(kept_sorted, slot, E * cap_pad)].set(
        sorted_tok, mode="drop")
    g = jnp.zeros((n_tok,), jnp.int32).at[sorted_tok].set(
        jnp.where(kept_sorted, slot, E * cap_pad + sorted_tok))

    xf_pad = jnp.concatenate([xf, jnp.zeros((1, d), xf.dtype)], axis=0)
    xg = xf_pad[d_flat]
    yg = _expert_matmul(xg, W_experts, b_experts, cap_pad).reshape(-1, d)
    table = jnp.concatenate([yg, xf], axis=0)
    out = table[g] * route_probs[:, None]
    return out.reshape(b, s, d)


# speedup vs baseline: 1.0977x; 1.0977x over previous
import functools

import numpy as np
import jax
import jax.numpy as jnp
from jax import lax
from jax.experimental import pallas as pl
from jax.experimental.pallas import tpu as pltpu
from jax.experimental.pallas import tpu_sc as plsc

_CAPACITY_FACTOR = 1.2
_ROW_BLK = 256


def _bits_masked(k0, k1, n, n_max):
    ji = jnp.arange(n_max, dtype=jnp.int32)
    half = (n + 1) // 2
    x0 = ji.astype(jnp.uint32)
    x1 = jnp.where(ji < (n // 2), ji + half, 0).astype(jnp.uint32)
    ks2 = k0 ^ k1 ^ jnp.uint32(0x1BD11BDA)
    ks = (k0, k1, ks2)
    v0 = x0 + ks[0]
    v1 = x1 + ks[1]
    rotations = ((13, 15, 26, 6), (17, 29, 16, 24))
    for i in range(5):
        for r in rotations[i % 2]:
            v0 = v0 + v1
            v1 = (v1 << jnp.uint32(r)) | (v1 >> jnp.uint32(32 - r))
            v1 = v0 ^ v1
        v0 = v0 + ks[(i + 1) % 3]
        v1 = v1 + ks[(i + 2) % 3] + jnp.uint32(i + 1)
    lo = v1[jnp.clip(ji - half, 0, n_max - 1)]
    return jnp.where(ji < half, v0, lo)


def _round_key_data(E, max_rounds):
    keys = [jax.random.fold_in(jax.random.key(1), i) for i in range(E)]
    rows = []
    for _ in range(max_rounds):
        subs = []
        for i in range(E):
            keys[i], sub = jax.random.split(keys[i])
            subs.append(jax.random.key_data(sub))
        rows.append(jnp.stack(subs))
    return jnp.stack(rows)


_NW = 32
_L = 16


def _sc_radix_sigma(keys_t, rows, n):
    V = n // _L
    vshift = V.bit_length() - 1
    mesh = plsc.VectorSubcoreMesh(core_axis_name="c", subcore_axis_name="s")

    @functools.partial(
        pl.kernel, mesh=mesh,
        out_type=jax.ShapeDtypeStruct((rows, n), jnp.int32),
        compiler_params=pltpu.CompilerParams(use_tc_tiling_on_sc=False,
                                             needs_layout_passes=False),
        scratch_types=[
            pltpu.VMEM((n,), jnp.int32),
            pltpu.VMEM((n,), jnp.int32),
            pltpu.VMEM((n,), jnp.int32),
            pltpu.VMEM((n,), jnp.int32),
            pltpu.VMEM((256 * _L,), jnp.int32),
        ],
    )
    def sort_kernel(keys_hbm, sig_hbm, ka, kb, pa, pb, offs):
        wid = lax.axis_index("s") * 2 + lax.axis_index("c")
        lanes = lax.iota(jnp.int32, 16)

        def radix_pass(p, src_k, src_p, dst_k, dst_p):
            shift = jnp.int32(8 * p)
            last = p == 3

            def zero_body(i, c):
                offs[pl.ds(i * _L, _L)] = jnp.zeros((_L,), jnp.int32)
                return c
            lax.fori_loop(0, 256, zero_body, 0)

            def count_body(v, c):
                kv = src_k[pl.ds(v * _L, _L)]
                d = lax.shift_right_logical(kv, shift) & jnp.int32(0xFF)
                addr = d * _L + lanes
                cur = plsc.load_gather(offs, [addr])
                plsc.store_scatter(offs, [addr], cur + 1)
                return c
            lax.fori_loop(0, V, count_body, 0)

            def scan_body(i, carry):
                h = offs[pl.ds(i * _L, _L)]
                offs[pl.ds(i * _L, _L)] = plsc.cumsum(h) - h + carry
                return carry + jnp.sum(h)
            lax.fori_loop(0, 256, scan_body, jnp.int32(0))

            def permute_body(v, c):
                kv = src_k[pl.ds(v * _L, _L)]
                pv = src_p[pl.ds(v * _L, _L)]
                d = lax.shift_right_logical(kv, shift) & jnp.int32(0xFF)
                addr = d * _L + lanes
                rho = plsc.load_gather(offs, [addr])
                plsc.store_scatter(offs, [addr], rho + 1)
                if last:
                    dest = rho
                else:
                    dest = ((rho & jnp.int32(V - 1)) * _L
                            + lax.shift_right_logical(rho, jnp.int32(vshift)))
                    plsc.store_scatter(dst_k, [dest], kv)
                plsc.store_scatter(dst_p, [dest], pv)
                return c
            lax.fori_loop(0, V, permute_body, 0)

        def do_row(ridx):
            pltpu.sync_copy(keys_hbm.at[ridx], ka)

            def init_body(v, c):
                pa[pl.ds(v * _L, _L)] = lanes * V + v
                return c
            lax.fori_loop(0, V, init_body, 0)

            radix_pass(0, ka, pa, kb, pb)
            radix_pass(1, kb, pb, ka, pa)
            radix_pass(2, ka, pa, kb, pb)
            radix_pass(3, kb, pb, ka, pa)
            pltpu.sync_copy(pa, sig_hbm.at[ridx])

        n_iter = (rows + _NW - 1) // _NW

        def outer_body(k, c):
            ridx = wid + k * _NW

            @pl.when(ridx < rows)
            def _():
                do_row(ridx)
            return c
        lax.fori_loop(0, n_iter, outer_body, 0)

    return sort_kernel(keys_t)


def _expert_matmul_kernel(x_ref, w_ref, b_ref, o_ref):
    acc = jax.lax.dot_general(
        x_ref[0], w_ref[0],
        dimension_numbers=(((1,), (1,)), ((), ())),
        preferred_element_type=jnp.float32,
    )
    o_ref[0] = acc + b_ref[0]


def _expert_matmul(xg, W_experts, b_experts, cap_pad):
    E, D = W_experts.shape[0], W_experts.shape[1]
    grid = (E, cap_pad // _ROW_BLK)
    return pl.pallas_call(
        _expert_matmul_kernel,
        grid=grid,
        in_specs=[
            pl.BlockSpec((1, _ROW_BLK, D), lambda i, c: (i, c, 0)),
            pl.BlockSpec((1, D, D), lambda i, c: (i, 0, 0)),
            pl.BlockSpec((1, 1, D), lambda i, c: (i, 0, 0)),
        ],
        out_specs=pl.BlockSpec((1, _ROW_BLK, D), lambda i, c: (i, c, 0)),
        out_shape=jax.ShapeDtypeStruct((E, cap_pad, D), jnp.float32),
    )(xg.reshape(E, cap_pad, D), W_experts, b_experts.reshape(E, 1, D))


def kernel(x, W_switch, b_switch, W_experts, b_experts):
    b, s, d = x.shape
    E = W_switch.shape[0]
    n_tok = b * s
    xf = x.reshape(-1, d)

    logits = xf @ W_switch.T + b_switch
    probs = jax.nn.softmax(logits, axis=-1)
    route_probs = jnp.max(probs, axis=-1)
    routes = jnp.argmax(probs, axis=-1).astype(jnp.int32)

    capacity = int(_CAPACITY_FACTOR * n_tok / E)
    cap_pad = ((capacity + _ROW_BLK - 1) // _ROW_BLK) * _ROW_BLK
    rounds_np = np.array([int(np.ceil(3 * np.log(max(1, t)) / np.log(2)))
                          for t in range(n_tok + 1)], dtype=np.int32)
    rounds_table = jnp.asarray(rounds_np)
    max_rounds = int(rounds_np.max())

    eids = jnp.arange(E, dtype=jnp.int32)
    counts = jnp.sum(routes[None, :] == eids[:, None], axis=1).astype(jnp.int32)
    num_rounds = rounds_table[counts]
    r_needed = jnp.max(num_rounds)

    rk = _round_key_data(E, max_rounds)
    ji = jnp.arange(n_tok, dtype=jnp.int32)

    kflat = rk.reshape(max_rounds * E, 2)
    nflat = jnp.tile(counts, max_rounds)
    bits = jax.vmap(lambda a, c, n: _bits_masked(a, c, n, n_tok))(
        kflat[:, 0], kflat[:, 1], nflat)
    active = ((ji[None, :] < nflat[:, None])
              & (jnp.repeat(jnp.arange(max_rounds, dtype=jnp.int32), E)[:, None]
                 < jnp.tile(num_rounds, max_rounds)[:, None]))
    keys = jnp.where(active, bits, jnp.uint32(0xFFFFFFFF))
    n_rows = max_rounds * E
    keys_i = jax.lax.bitcast_convert_type(keys, jnp.int32)
    keys_t = keys_i.reshape(n_rows, _L, n_tok // _L).transpose(0, 2, 1)
    sigma = _sc_radix_sigma(keys_t.reshape(n_rows, n_tok), n_rows, n_tok)

    sig = sigma.reshape(max_rounds, E, n_tok)
    r_cur = max_rounds
    while r_cur > 1:
        half = r_cur // 2
        a = sig[0:2 * half:2]
        bb = sig[1:2 * half:2]
        comp = jnp.take_along_axis(a, bb, axis=2)
        if r_cur % 2:
            comp = jnp.concatenate([comp, sig[2 * half:]], axis=0)
        sig = comp
        r_cur = half + (r_cur % 2)
    perm = sig[0]

    rowi = jnp.broadcast_to(eids[:, None], (E, n_tok))
    colj = jnp.broadcast_to(ji[None, :], (E, n_tok))
    inv = jnp.zeros((E, n_tok), jnp.int32).at[rowi, perm].set(colj)
    keep_rank = (counts[:, None] <= capacity) | (inv < capacity)

    sorted_tok = jnp.argsort(routes, stable=True).astype(jnp.int32)
    e_sorted = routes[sorted_tok]
    starts = jnp.concatenate([jnp.zeros((1,), jnp.int32),
                              jnp.cumsum(counts)[:-1].astype(jnp.int32)])
    pos = ji - starts[e_sorted]
    kept_sorted = keep_rank[e_sorted, pos]

    kept_count = jnp.minimum(counts, capacity)
    kept_before = jnp.concatenate([jnp.zeros((1,), jnp.int32),
                                   jnp.cumsum(kept_count)[:-1].astype(jnp.int32)])
    kc = jnp.cumsum(kept_sorted.astype(jnp.int32))
    slot = e_sorted * cap_pad + (kc - 1 - kept_before[e_sorted])

    d_flat = jnp.full((E * cap_pad,), n_tok, jnp.int32)
    d_flat = d_flat.at[jnp.where(kept_sorted, slot, E * cap_pad)].set(
        sorted_tok, mode="drop")
    g = jnp.zeros((n_tok,), jnp.int32).at[sorted_tok].set(
        jnp.where(kept_sorted, slot, E * cap_pad + sorted_tok))

    xf_pad = jnp.concatenate([xf, jnp.zeros((1, d), xf.dtype)], axis=0)
    xg = xf_pad[d_flat]
    yg = _expert_matmul(xg, W_experts, b_experts, cap_pad).reshape(-1, d)
    table = jnp.concatenate([yg, xf], axis=0)
    out = table[g] * route_probs[:, None]
    return out.reshape(b, s, d)


# speedup vs baseline: 1.1592x; 1.0560x over previous
import functools

import numpy as np
import jax
import jax.numpy as jnp
from jax import lax
from jax.experimental import pallas as pl
from jax.experimental.pallas import tpu as pltpu
from jax.experimental.pallas import tpu_sc as plsc

_CAPACITY_FACTOR = 1.2
_ROW_BLK = 256


def _bits_masked(k0, k1, n, n_max):
    ji = jnp.arange(n_max, dtype=jnp.int32)
    half = (n + 1) // 2
    x0 = ji.astype(jnp.uint32)
    x1 = jnp.where(ji < (n // 2), ji + half, 0).astype(jnp.uint32)
    ks2 = k0 ^ k1 ^ jnp.uint32(0x1BD11BDA)
    ks = (k0, k1, ks2)
    v0 = x0 + ks[0]
    v1 = x1 + ks[1]
    rotations = ((13, 15, 26, 6), (17, 29, 16, 24))
    for i in range(5):
        for r in rotations[i % 2]:
            v0 = v0 + v1
            v1 = (v1 << jnp.uint32(r)) | (v1 >> jnp.uint32(32 - r))
            v1 = v0 ^ v1
        v0 = v0 + ks[(i + 1) % 3]
        v1 = v1 + ks[(i + 2) % 3] + jnp.uint32(i + 1)
    lo = v1[jnp.clip(ji - half, 0, n_max - 1)]
    return jnp.where(ji < half, v0, lo)


def _round_key_data(E, max_rounds):
    keys = [jax.random.fold_in(jax.random.key(1), i) for i in range(E)]
    rows = []
    for _ in range(max_rounds):
        subs = []
        for i in range(E):
            keys[i], sub = jax.random.split(keys[i])
            subs.append(jax.random.key_data(sub))
        rows.append(jnp.stack(subs))
    return jnp.stack(rows)


_NW = 32
_L = 16


def _sc_radix_sigma(keys_t, nr16, rows, n, n_exp):
    V = n // _L
    vshift = V.bit_length() - 1
    mesh = plsc.VectorSubcoreMesh(core_axis_name="c", subcore_axis_name="s")

    @functools.partial(
        pl.kernel, mesh=mesh,
        out_type=jax.ShapeDtypeStruct((rows, n), jnp.int32),
        compiler_params=pltpu.CompilerParams(use_tc_tiling_on_sc=False,
                                             needs_layout_passes=False),
        scratch_types=[
            pltpu.VMEM((n,), jnp.int32),
            pltpu.VMEM((n,), jnp.int32),
            pltpu.VMEM((n,), jnp.int32),
            pltpu.VMEM((n,), jnp.int32),
            pltpu.VMEM((256 * _L,), jnp.int32),
            pltpu.VMEM((n,), jnp.int32),
            pltpu.VMEM((_L,), jnp.int32),
        ],
    )
    def sort_kernel(keys_hbm, nr_hbm, sig_hbm, ka, kb, pa, pb, offs, occ,
                    nr_v):
        wid = lax.axis_index("s") * 2 + lax.axis_index("c")
        lanes = lax.iota(jnp.int32, 16)
        pltpu.sync_copy(nr_hbm, nr_v)

        def radix_pass(p, src_k, src_p, dst_k, dst_p):
            shift = jnp.int32(8 * p)
            last = p == 3

            def zero_body(i, c):
                offs[pl.ds(i * 4 * _L, _L)] = jnp.zeros((_L,), jnp.int32)
                offs[pl.ds((i * 4 + 1) * _L, _L)] = jnp.zeros((_L,), jnp.int32)
                offs[pl.ds((i * 4 + 2) * _L, _L)] = jnp.zeros((_L,), jnp.int32)
                offs[pl.ds((i * 4 + 3) * _L, _L)] = jnp.zeros((_L,), jnp.int32)
                return c
            lax.fori_loop(0, 64, zero_body, 0)

            def count_one(v):
                kv = src_k[pl.ds(v * _L, _L)]
                d = lax.shift_right_logical(kv, shift) & jnp.int32(0xFF)
                addr = d * _L + lanes
                cur = plsc.load_gather(offs, [addr])
                plsc.store_scatter(offs, [addr], cur + 1)
                occ[pl.ds(v * _L, _L)] = cur

            def count_body(u, c):
                count_one(2 * u)
                count_one(2 * u + 1)
                return c
            lax.fori_loop(0, V // 2, count_body, 0)

            def scan_body(i, carry):
                h = offs[pl.ds(i * _L, _L)]
                offs[pl.ds(i * _L, _L)] = plsc.cumsum(h) - h + carry
                return carry + jnp.sum(h)
            lax.fori_loop(0, 256, scan_body, jnp.int32(0))

            def permute_one(v):
                kv = src_k[pl.ds(v * _L, _L)]
                pv = src_p[pl.ds(v * _L, _L)]
                d = lax.shift_right_logical(kv, shift) & jnp.int32(0xFF)
                addr = d * _L + lanes
                rho = plsc.load_gather(offs, [addr]) + occ[pl.ds(v * _L, _L)]
                if last:
                    dest = rho
                else:
                    dest = ((rho & jnp.int32(V - 1)) * _L
                            + lax.shift_right_logical(rho, jnp.int32(vshift)))
                    plsc.store_scatter(dst_k, [dest], kv)
                plsc.store_scatter(dst_p, [dest], pv)

            def permute_body(u, c):
                permute_one(2 * u)
                permute_one(2 * u + 1)
                return c
            lax.fori_loop(0, V // 2, permute_body, 0)

        def full_sort(ridx):
            pltpu.sync_copy(keys_hbm.at[ridx], ka)

            def init_body(v, c):
                pa[pl.ds(2 * v * _L, _L)] = lanes * V + 2 * v
                pa[pl.ds((2 * v + 1) * _L, _L)] = lanes * V + 2 * v + 1
                return c
            lax.fori_loop(0, V // 2, init_body, 0)

            radix_pass(0, ka, pa, kb, pb)
            radix_pass(1, kb, pb, ka, pa)
            radix_pass(2, ka, pa, kb, pb)
            radix_pass(3, kb, pb, ka, pa)
            pltpu.sync_copy(pa, sig_hbm.at[ridx])

        def identity_row(ridx):
            def init_body(v, c):
                pa[pl.ds(2 * v * _L, _L)] = lanes + 2 * v * _L
                pa[pl.ds((2 * v + 1) * _L, _L)] = lanes + (2 * v + 1) * _L
                return c
            lax.fori_loop(0, V // 2, init_body, 0)
            pltpu.sync_copy(pa, sig_hbm.at[ridx])

        n_iter = (rows + _NW - 1) // _NW

        def outer_body(k, c):
            ridx = wid + k * _NW

            @pl.when(ridx < rows)
            def _():
                r = lax.div(ridx, n_exp)
                i = lax.rem(ridx, n_exp)
                nri = jnp.sum(jnp.where(lanes == i, nr_v[...], 0))

                @pl.when(r < nri)
                def _():
                    full_sort(ridx)

                @pl.when(r >= nri)
                def _():
                    identity_row(ridx)
            return c
        lax.fori_loop(0, n_iter, outer_body, 0)

    return sort_kernel(keys_t, nr16)


def _expert_matmul_kernel(x_ref, w_ref, b_ref, o_ref):
    acc = jax.lax.dot_general(
        x_ref[0], w_ref[0],
        dimension_numbers=(((1,), (1,)), ((), ())),
        preferred_element_type=jnp.float32,
    )
    o_ref[0] = acc + b_ref[0]


def _expert_matmul(xg, W_experts, b_experts, cap_pad):
    E, D = W_experts.shape[0], W_experts.shape[1]
    grid = (E, cap_pad // _ROW_BLK)
    return pl.pallas_call(
        _expert_matmul_kernel,
        grid=grid,
        in_specs=[
            pl.BlockSpec((1, _ROW_BLK, D), lambda i, c: (i, c, 0)),
            pl.BlockSpec((1, D, D), lambda i, c: (i, 0, 0)),
            pl.BlockSpec((1, 1, D), lambda i, c: (i, 0, 0)),
        ],
        out_specs=pl.BlockSpec((1, _ROW_BLK, D), lambda i, c: (i, c, 0)),
        out_shape=jax.ShapeDtypeStruct((E, cap_pad, D), jnp.float32),
    )(xg.reshape(E, cap_pad, D), W_experts, b_experts.reshape(E, 1, D))


def kernel(x, W_switch, b_switch, W_experts, b_experts):
    b, s, d = x.shape
    E = W_switch.shape[0]
    n_tok = b * s
    xf = x.reshape(-1, d)

    logits = xf @ W_switch.T + b_switch
    probs = jax.nn.softmax(logits, axis=-1)
    route_probs = jnp.max(probs, axis=-1)
    routes = jnp.argmax(probs, axis=-1).astype(jnp.int32)

    capacity = int(_CAPACITY_FACTOR * n_tok / E)
    cap_pad = ((capacity + _ROW_BLK - 1) // _ROW_BLK) * _ROW_BLK
    rounds_np = np.array([int(np.ceil(3 * np.log(max(1, t)) / np.log(2)))
                          for t in range(n_tok + 1)], dtype=np.int32)
    rounds_table = jnp.asarray(rounds_np)
    max_rounds = int(rounds_np.max())

    eids = jnp.arange(E, dtype=jnp.int32)
    counts = jnp.sum(routes[None, :] == eids[:, None], axis=1).astype(jnp.int32)
    num_rounds = rounds_table[counts]
    r_needed = jnp.max(num_rounds)

    rk = _round_key_data(E, max_rounds)
    ji = jnp.arange(n_tok, dtype=jnp.int32)

    kflat = rk.reshape(max_rounds * E, 2)
    nflat = jnp.tile(counts, max_rounds)
    bits = jax.vmap(lambda a, c, n: _bits_masked(a, c, n, n_tok))(
        kflat[:, 0], kflat[:, 1], nflat)
    active = ((ji[None, :] < nflat[:, None])
              & (jnp.repeat(jnp.arange(max_rounds, dtype=jnp.int32), E)[:, None]
                 < jnp.tile(num_rounds, max_rounds)[:, None]))
    keys = jnp.where(active, bits, jnp.uint32(0xFFFFFFFF))
    n_rows = max_rounds * E
    keys_i = jax.lax.bitcast_convert_type(keys, jnp.int32)
    keys_t = keys_i.reshape(n_rows, _L, n_tok // _L).transpose(0, 2, 1)
    nr16 = jnp.zeros((_L,), jnp.int32).at[:E].set(num_rounds)
    sigma = _sc_radix_sigma(keys_t.reshape(n_rows, n_tok), nr16,
                            n_rows, n_tok, E)

    sig = sigma.reshape(max_rounds, E, n_tok)
    r_cur = max_rounds
    while r_cur > 1:
        half = r_cur // 2
        a = sig[0:2 * half:2]
        bb = sig[1:2 * half:2]
        comp = jnp.take_along_axis(a, bb, axis=2)
        if r_cur % 2:
            comp = jnp.concatenate([comp, sig[2 * half:]], axis=0)
        sig = comp
        r_cur = half + (r_cur % 2)
    perm = sig[0]

    rowi = jnp.broadcast_to(eids[:, None], (E, n_tok))
    colj = jnp.broadcast_to(ji[None, :], (E, n_tok))
    inv = jnp.zeros((E, n_tok), jnp.int32).at[rowi, perm].set(colj)
    keep_rank = (counts[:, None] <= capacity) | (inv < capacity)

    sorted_tok = jnp.argsort(routes, stable=True).astype(jnp.int32)
    e_sorted = routes[sorted_tok]
    starts = jnp.concatenate([jnp.zeros((1,), jnp.int32),
                              jnp.cumsum(counts)[:-1].astype(jnp.int32)])
    pos = ji - starts[e_sorted]
    kept_sorted = keep_rank[e_sorted, pos]

    kept_count = jnp.minimum(counts, capacity)
    kept_before = jnp.concatenate([jnp.zeros((1,), jnp.int32),
                                   jnp.cumsum(kept_count)[:-1].astype(jnp.int32)])
    kc = jnp.cumsum(kept_sorted.astype(jnp.int32))
    slot = e_sorted * cap_pad + (kc - 1 - kept_before[e_sorted])

    d_flat = jnp.full((E * cap_pad,), n_tok, jnp.int32)
    d_flat = d_flat.at[jnp.where(kept_sorted, slot, E * cap_pad)].set(
        sorted_tok, mode="drop")
    g = jnp.zeros((n_tok,), jnp.int32).at[sorted_tok].set(
        jnp.where(kept_sorted, slot, E * cap_pad + sorted_tok))

    xf_pad = jnp.concatenate([xf, jnp.zeros((1, d), xf.dtype)], axis=0)
    xg = xf_pad[d_flat]
    yg = _expert_matmul(xg, W_experts, b_experts, cap_pad).reshape(-1, d)
    table = jnp.concatenate([yg, xf], axis=0)
    out = table[g] * route_probs[:, None]
    return out.reshape(b, s, d)


# speedup vs baseline: 1.3434x; 1.1590x over previous
import functools

import numpy as np
import jax
import jax.numpy as jnp
from jax import lax
from jax.experimental import pallas as pl
from jax.experimental.pallas import tpu as pltpu
from jax.experimental.pallas import tpu_sc as plsc

_CAPACITY_FACTOR = 1.2
_ROW_BLK = 256


def _bits_masked(k0, k1, n, n_max):
    ji = jnp.arange(n_max, dtype=jnp.int32)
    half = (n + 1) // 2
    x0 = ji.astype(jnp.uint32)
    x1 = jnp.where(ji < (n // 2), ji + half, 0).astype(jnp.uint32)
    ks2 = k0 ^ k1 ^ jnp.uint32(0x1BD11BDA)
    ks = (k0, k1, ks2)
    v0 = x0 + ks[0]
    v1 = x1 + ks[1]
    rotations = ((13, 15, 26, 6), (17, 29, 16, 24))
    for i in range(5):
        for r in rotations[i % 2]:
            v0 = v0 + v1
            v1 = (v1 << jnp.uint32(r)) | (v1 >> jnp.uint32(32 - r))
            v1 = v0 ^ v1
        v0 = v0 + ks[(i + 1) % 3]
        v1 = v1 + ks[(i + 2) % 3] + jnp.uint32(i + 1)
    lo = v1[jnp.clip(ji - half, 0, n_max - 1)]
    return jnp.where(ji < half, v0, lo)


def _round_key_data(E, max_rounds):
    keys = [jax.random.fold_in(jax.random.key(1), i) for i in range(E)]
    rows = []
    for _ in range(max_rounds):
        subs = []
        for i in range(E):
            keys[i], sub = jax.random.split(keys[i])
            subs.append(jax.random.key_data(sub))
        rows.append(jnp.stack(subs))
    return jnp.stack(rows)


_NW = 32
_L = 16


def _sc_radix_sigma(keys_t, nr16, rows, n, n_exp):
    V = n // _L
    vshift = V.bit_length() - 1
    mesh = plsc.VectorSubcoreMesh(core_axis_name="c", subcore_axis_name="s")

    @functools.partial(
        pl.kernel, mesh=mesh,
        out_type=jax.ShapeDtypeStruct((rows, n), jnp.int32),
        compiler_params=pltpu.CompilerParams(use_tc_tiling_on_sc=False,
                                             needs_layout_passes=False),
        scratch_types=[
            pltpu.VMEM((n,), jnp.int32),
            pltpu.VMEM((n,), jnp.int32),
            pltpu.VMEM((n,), jnp.int32),
            pltpu.VMEM((n,), jnp.int32),
            pltpu.VMEM((256 * _L,), jnp.int32),
            pltpu.VMEM((n,), jnp.int32),
            pltpu.VMEM((_L,), jnp.int32),
        ],
    )
    def sort_kernel(keys_hbm, nr_hbm, sig_hbm, ka, kb, pa, pb, offs, occ,
                    nr_v):
        wid = lax.axis_index("s") * 2 + lax.axis_index("c")
        lanes = lax.iota(jnp.int32, 16)
        pltpu.sync_copy(nr_hbm, nr_v)

        def radix_pass(p, src_k, src_p, dst_k, dst_p):
            shift = jnp.int32(8 * p)
            last = p == 3

            def zero_body(i, c):
                offs[pl.ds(i * 4 * _L, _L)] = jnp.zeros((_L,), jnp.int32)
                offs[pl.ds((i * 4 + 1) * _L, _L)] = jnp.zeros((_L,), jnp.int32)
                offs[pl.ds((i * 4 + 2) * _L, _L)] = jnp.zeros((_L,), jnp.int32)
                offs[pl.ds((i * 4 + 3) * _L, _L)] = jnp.zeros((_L,), jnp.int32)
                return c
            lax.fori_loop(0, 64, zero_body, 0)

            def count_one(v):
                kv = src_k[pl.ds(v * _L, _L)]
                d = lax.shift_right_logical(kv, shift) & jnp.int32(0xFF)
                addr = d * _L + lanes
                cur = plsc.load_gather(offs, [addr])
                plsc.store_scatter(offs, [addr], cur + 1)
                occ[pl.ds(v * _L, _L)] = cur

            def count_body(u, c):
                count_one(2 * u)
                count_one(2 * u + 1)
                return c
            lax.fori_loop(0, V // 2, count_body, 0)

            def scan_body(i, carry):
                h = offs[pl.ds(i * _L, _L)]
                offs[pl.ds(i * _L, _L)] = plsc.cumsum(h) - h + carry
                return carry + jnp.sum(h)
            lax.fori_loop(0, 256, scan_body, jnp.int32(0))

            def permute_one(v):
                kv = src_k[pl.ds(v * _L, _L)]
                pv = src_p[pl.ds(v * _L, _L)]
                d = lax.shift_right_logical(kv, shift) & jnp.int32(0xFF)
                addr = d * _L + lanes
                rho = plsc.load_gather(offs, [addr]) + occ[pl.ds(v * _L, _L)]
                if last:
                    dest = rho
                else:
                    dest = ((rho & jnp.int32(V - 1)) * _L
                            + lax.shift_right_logical(rho, jnp.int32(vshift)))
                    plsc.store_scatter(dst_k, [dest], kv)
                plsc.store_scatter(dst_p, [dest], pv)

            def permute_body(u, c):
                permute_one(2 * u)
                permute_one(2 * u + 1)
                return c
            lax.fori_loop(0, V // 2, permute_body, 0)

        def full_sort(ridx):
            pltpu.sync_copy(keys_hbm.at[ridx], ka)

            def init_body(v, c):
                pa[pl.ds(2 * v * _L, _L)] = lanes * V + 2 * v
                pa[pl.ds((2 * v + 1) * _L, _L)] = lanes * V + 2 * v + 1
                return c
            lax.fori_loop(0, V // 2, init_body, 0)

            radix_pass(0, ka, pa, kb, pb)
            radix_pass(1, kb, pb, ka, pa)
            radix_pass(2, ka, pa, kb, pb)
            radix_pass(3, kb, pb, ka, pa)
            pltpu.sync_copy(pa, sig_hbm.at[ridx])

        def identity_row(ridx):
            def init_body(v, c):
                pa[pl.ds(2 * v * _L, _L)] = lanes + 2 * v * _L
                pa[pl.ds((2 * v + 1) * _L, _L)] = lanes + (2 * v + 1) * _L
                return c
            lax.fori_loop(0, V // 2, init_body, 0)
            pltpu.sync_copy(pa, sig_hbm.at[ridx])

        n_iter = (rows + _NW - 1) // _NW

        def outer_body(k, c):
            ridx = wid + k * _NW

            @pl.when(ridx < rows)
            def _():
                r = lax.div(ridx, n_exp)
                i = lax.rem(ridx, n_exp)
                nri = jnp.sum(jnp.where(lanes == i, nr_v[...], 0))

                @pl.when(r < nri)
                def _():
                    full_sort(ridx)

                @pl.when(r >= nri)
                def _():
                    identity_row(ridx)
            return c
        lax.fori_loop(0, n_iter, outer_body, 0)

    return sort_kernel(keys_t, nr16)


def _sc_compose_kept(sigma, nr16, rows, n, n_exp, capacity):
    V = n // _L
    mesh = plsc.VectorSubcoreMesh(core_axis_name="c", subcore_axis_name="s")

    @functools.partial(
        pl.kernel, mesh=mesh,
        out_type=jax.ShapeDtypeStruct((n_exp, n), jnp.int32),
        compiler_params=pltpu.CompilerParams(use_tc_tiling_on_sc=False,
                                             needs_layout_passes=False),
        scratch_types=[
            pltpu.VMEM((n,), jnp.int32),
            pltpu.VMEM((n,), jnp.int32),
            pltpu.VMEM((n,), jnp.int32),
            pltpu.VMEM((_L,), jnp.int32),
        ],
    )
    def compose_kernel(sig_hbm, nr_hbm, mask_hbm, srow, g, mask, nr_v):
        wid = lax.axis_index("s") * 2 + lax.axis_index("c")
        lanes = lax.iota(jnp.int32, 16)
        pltpu.sync_copy(nr_hbm, nr_v)

        @pl.when(wid < n_exp)
        def _():
            i = wid
            nri = jnp.sum(jnp.where(lanes == i, nr_v[...], 0))

            def zero_body(v, c):
                mask[pl.ds(v * _L, _L)] = jnp.zeros((_L,), jnp.int32)
                return c
            lax.fori_loop(0, V, zero_body, 0)

            @pl.when(nri > 0)
            def _():
                pltpu.sync_copy(sig_hbm.at[(nri - 1) * n_exp + i], g)

                def round_body(k, c):
                    r = nri - 2 - k
                    pltpu.sync_copy(sig_hbm.at[r * n_exp + i], srow)

                    def gather_body(v, c2):
                        gv = g[pl.ds(v * _L, _L)]
                        g[pl.ds(v * _L, _L)] = plsc.load_gather(srow, [gv])
                        return c2
                    lax.fori_loop(0, V, gather_body, 0)
                    return c
                lax.fori_loop(0, nri - 1, round_body, 0)

                nfull = capacity // _L
                ntail = capacity - nfull * _L

                def mark_body(v, c):
                    gv = g[pl.ds(v * _L, _L)]
                    plsc.store_scatter(mask, [gv], jnp.ones((_L,), jnp.int32))
                    return c
                lax.fori_loop(0, nfull, mark_body, 0)
                if ntail:
                    gv = g[pl.ds(nfull * _L, _L)]
                    plsc.store_scatter(mask, [gv],
                                       jnp.ones((_L,), jnp.int32),
                                       mask=lanes < ntail)

            pltpu.sync_copy(mask, mask_hbm.at[i])

    return compose_kernel(sigma, nr16)


def _expert_matmul_kernel(x_ref, w_ref, b_ref, o_ref):
    acc = jax.lax.dot_general(
        x_ref[0], w_ref[0],
        dimension_numbers=(((1,), (1,)), ((), ())),
        preferred_element_type=jnp.float32,
    )
    o_ref[0] = acc + b_ref[0]


def _expert_matmul(xg, W_experts, b_experts, cap_pad):
    E, D = W_experts.shape[0], W_experts.shape[1]
    grid = (E, cap_pad // _ROW_BLK)
    return pl.pallas_call(
        _expert_matmul_kernel,
        grid=grid,
        in_specs=[
            pl.BlockSpec((1, _ROW_BLK, D), lambda i, c: (i, c, 0)),
            pl.BlockSpec((1, D, D), lambda i, c: (i, 0, 0)),
            pl.BlockSpec((1, 1, D), lambda i, c: (i, 0, 0)),
        ],
        out_specs=pl.BlockSpec((1, _ROW_BLK, D), lambda i, c: (i, c, 0)),
        out_shape=jax.ShapeDtypeStruct((E, cap_pad, D), jnp.float32),
    )(xg.reshape(E, cap_pad, D), W_experts, b_experts.reshape(E, 1, D))


def kernel(x, W_switch, b_switch, W_experts, b_experts):
    b, s, d = x.shape
    E = W_switch.shape[0]
    n_tok = b * s
    xf = x.reshape(-1, d)

    logits = xf @ W_switch.T + b_switch
    probs = jax.nn.softmax(logits, axis=-1)
    route_probs = jnp.max(probs, axis=-1)
    routes = jnp.argmax(probs, axis=-1).astype(jnp.int32)

    capacity = int(_CAPACITY_FACTOR * n_tok / E)
    cap_pad = ((capacity + _ROW_BLK - 1) // _ROW_BLK) * _ROW_BLK
    rounds_np = np.array([int(np.ceil(3 * np.log(max(1, t)) / np.log(2)))
                          for t in range(n_tok + 1)], dtype=np.int32)
    rounds_table = jnp.asarray(rounds_np)
    max_rounds = int(rounds_np.max())

    eids = jnp.arange(E, dtype=jnp.int32)
    counts = jnp.sum(routes[None, :] == eids[:, None], axis=1).astype(jnp.int32)
    num_rounds = rounds_table[counts]
    r_needed = jnp.max(num_rounds)

    rk = _round_key_data(E, max_rounds)
    ji = jnp.arange(n_tok, dtype=jnp.int32)

    kflat = rk.reshape(max_rounds * E, 2)
    nflat = jnp.tile(counts, max_rounds)
    bits = jax.vmap(lambda a, c, n: _bits_masked(a, c, n, n_tok))(
        kflat[:, 0], kflat[:, 1], nflat)
    active = ((ji[None, :] < nflat[:, None])
              & (jnp.repeat(jnp.arange(max_rounds, dtype=jnp.int32), E)[:, None]
                 < jnp.tile(num_rounds, max_rounds)[:, None]))
    keys = jnp.where(active, bits, jnp.uint32(0xFFFFFFFF))
    n_rows = max_rounds * E
    keys_i = jax.lax.bitcast_convert_type(keys, jnp.int32)
    keys_t = keys_i.reshape(n_rows, _L, n_tok // _L).transpose(0, 2, 1)
    nr16 = jnp.zeros((_L,), jnp.int32).at[:E].set(num_rounds)
    sigma = _sc_radix_sigma(keys_t.reshape(n_rows, n_tok), nr16,
                            n_rows, n_tok, E)

    kept_mask = _sc_compose_kept(sigma, nr16, n_rows, n_tok, E, capacity)
    keep_rank = (counts[:, None] <= capacity) | (kept_mask > 0)

    sorted_tok = jnp.argsort(routes, stable=True).astype(jnp.int32)
    e_sorted = routes[sorted_tok]
    starts = jnp.concatenate([jnp.zeros((1,), jnp.int32),
                              jnp.cumsum(counts)[:-1].astype(jnp.int32)])
    pos = ji - starts[e_sorted]
    kept_sorted = keep_rank[e_sorted, pos]

    kept_count = jnp.minimum(counts, capacity)
    kept_before = jnp.concatenate([jnp.zeros((1,), jnp.int32),
                                   jnp.cumsum(kept_count)[:-1].astype(jnp.int32)])
    kc = jnp.cumsum(kept_sorted.astype(jnp.int32))
    slot = e_sorted * cap_pad + (kc - 1 - kept_before[e_sorted])

    d_flat = jnp.arange(E * cap_pad, dtype=jnp.int32) % n_tok
    d_flat = d_flat.at[jnp.where(kept_sorted, slot, E * cap_pad)].set(
        sorted_tok, mode="drop")
    g = jnp.zeros((n_tok,), jnp.int32).at[sorted_tok].set(
        jnp.where(kept_sorted, slot, E * cap_pad + sorted_tok))

    xg = xf[d_flat]
    yg = _expert_matmul(xg, W_experts, b_experts, cap_pad).reshape(-1, d)
    table = jnp.concatenate([yg, xf], axis=0)
    out = table[g] * route_probs[:, None]
    return out.reshape(b, s, d)
